# baseline (speedup 1.0000x reference)
import jax
import jax.numpy as jnp
from jax import lax
from jax.experimental import pallas as pl
from jax.experimental.pallas import tpu as pltpu
from jax.experimental.pallas import tpu_sc as plsc

NC = 2
NS = 16
NW = NC * NS
LANES = 16
CHUNK = 128
NBUF = 2


def _sc_body(idx_hbm, emb_hbm, out_hbm,
             idx_v, ro0, rd0, ro1, rd1,
             acc_v, sem0, sem1):
  n_chunks = idx_hbm.shape[2]
  n_steps = 2 * n_chunks

  wid = lax.axis_index("s") * NC + lax.axis_index("c")

  row_bufs = [(ro0, rd0), (ro1, rd1)]
  sems = [sem0, sem1]

  pltpu.sync_copy(idx_hbm.at[wid], idx_v)

  zero = jnp.zeros((LANES,), jnp.float32)
  for r in range(6):
    for j in range(8):
      acc_v[r, pl.ds(j * LANES, LANES)] = zero

  def start(s, b):
    sm = lax.rem(s, n_steps)
    g2 = lax.div(sm, n_chunks) * 2
    c = lax.rem(sm, n_chunks)
    ro, rd = row_bufs[b]
    pltpu.async_copy(emb_hbm.at[idx_v.at[g2, c]], ro, sems[b])
    pltpu.async_copy(emb_hbm.at[idx_v.at[g2 + 1, c]], rd, sems[b])

  start(0, 0)

  def step(s, b):
    start(s + 1, 1 - b)
    ro, rd = row_bufs[b]
    pltpu.make_async_copy(emb_hbm.at[idx_v.at[0, 0]], ro, sems[b]).wait()
    pltpu.make_async_copy(emb_hbm.at[idx_v.at[0, 0]], rd, sems[b]).wait()

    def body(i2, carry):
      a = list(carry)
      for u in range(2):
        i = 2 * i2 + u
        for j in range(8):
          o = ro[i, pl.ds(j * LANES, LANES)]
          d = rd[i, pl.ds(j * LANES, LANES)]
          a[3 * j + 0] = a[3 * j + 0] + o * d
          a[3 * j + 1] = a[3 * j + 1] + o * o
          a[3 * j + 2] = a[3 * j + 2] + d * d
      return tuple(a)

    accs = lax.fori_loop(0, CHUNK // 2, body, tuple(zero for _ in range(24)))

    row0 = lax.div(s, n_chunks) * 3
    for k in range(3):
      for j in range(8):
        sl = pl.ds(j * LANES, LANES)
        acc_v[row0 + k, sl] = acc_v[row0 + k, sl] + accs[3 * j + k]

  def loop_body(t, _):
    step(2 * t, 0)
    step(2 * t + 1, 1)
    return 0

  lax.fori_loop(0, n_steps // 2, loop_body, 0)

  ro, rd = row_bufs[0]
  pltpu.make_async_copy(emb_hbm.at[idx_v.at[0, 0]], ro, sems[0]).wait()
  pltpu.make_async_copy(emb_hbm.at[idx_v.at[0, 0]], rd, sems[0]).wait()

  pltpu.sync_copy(acc_v, out_hbm.at[wid])


def kernel(pos_pair, neg_pair, emb):
  B = pos_pair.shape[0]
  per_w = B // NW
  n_chunks = per_w // CHUNK
  idx = jnp.stack([pos_pair[:, 0], pos_pair[:, 1],
                   neg_pair[:, 0], neg_pair[:, 1]]).astype(jnp.int32)
  idx = idx.reshape(4, NW, n_chunks, CHUNK).transpose(1, 0, 2, 3)

  mesh = plsc.VectorSubcoreMesh(core_axis_name="c", subcore_axis_name="s",
                                num_cores=NC, num_subcores=NS)
  partials = pl.kernel(
      _sc_body,
      out_type=jax.ShapeDtypeStruct((NW, 6, 128), jnp.float32),
      mesh=mesh,
      scratch_types=[
          pltpu.VMEM((4, n_chunks, CHUNK), jnp.int32),
          pltpu.VMEM((CHUNK, 128), jnp.float32),
          pltpu.VMEM((CHUNK, 128), jnp.float32),
          pltpu.VMEM((CHUNK, 128), jnp.float32),
          pltpu.VMEM((CHUNK, 128), jnp.float32),
          pltpu.VMEM((6, 128), jnp.float32),
          pltpu.SemaphoreType.DMA,
          pltpu.SemaphoreType.DMA,
      ],
  )(idx, emb)

  p = jnp.sum(partials, axis=0)
  eps = jnp.float32(1e-8)

  def cos(num, so, sd):
    return num / (jnp.maximum(jnp.sqrt(so), eps) * jnp.maximum(jnp.sqrt(sd), eps))

  pos_dist = cos(p[0], p[1], p[2])
  neg_dist = cos(p[3], p[4], p[5])
  return (pos_dist, neg_dist)


# speedup vs baseline: 1.0374x; 1.0374x over previous
import jax
import jax.numpy as jnp
from jax import lax
from jax.experimental import pallas as pl
from jax.experimental.pallas import tpu as pltpu
from jax.experimental.pallas import tpu_sc as plsc

NC = 2
NS = 16
NW = NC * NS
LANES = 16
CHUNK = 64
NBUF = 6


def _sc_body(idx_hbm, emb_hbm, out_hbm, idx_v, *rest):
  row_bufs = [(rest[2 * b], rest[2 * b + 1]) for b in range(NBUF)]
  acc_v = rest[2 * NBUF]
  sems = rest[2 * NBUF + 1:]
  n_chunks = idx_hbm.shape[2]

  wid = lax.axis_index("s") * NC + lax.axis_index("c")

  pltpu.sync_copy(idx_hbm.at[wid], idx_v)

  steps = [(g, c) for g in range(2) for c in range(n_chunks)]

  def start(s):
    g, c = steps[s]
    b = s % NBUF
    ro, rd = row_bufs[b]
    ho = pltpu.async_copy(emb_hbm.at[idx_v.at[2 * g, c]], ro, sems[b])
    hd = pltpu.async_copy(emb_hbm.at[idx_v.at[2 * g + 1, c]], rd, sems[b])
    return (ho, hd)

  inflight = {s: start(s) for s in range(min(NBUF - 1, len(steps)))}

  zero = jnp.zeros((LANES,), jnp.float32)
  for g in range(2):
    accs = tuple(zero for _ in range(24))
    for c in range(n_chunks):
      s = g * n_chunks + c
      nxt = s + NBUF - 1
      if nxt < len(steps):
        inflight[nxt] = start(nxt)
      ho, hd = inflight.pop(s)
      ho.wait()
      hd.wait()
      ro, rd = row_bufs[s % NBUF]

      def body(i2, carry, ro=ro, rd=rd):
        a = list(carry)
        for u in range(2):
          i = 2 * i2 + u
          for j in range(8):
            o = ro[i, pl.ds(j * LANES, LANES)]
            d = rd[i, pl.ds(j * LANES, LANES)]
            a[3 * j + 0] = a[3 * j + 0] + o * d
            a[3 * j + 1] = a[3 * j + 1] + o * o
            a[3 * j + 2] = a[3 * j + 2] + d * d
        return tuple(a)

      accs = lax.fori_loop(0, CHUNK // 2, body, accs)

    for j in range(8):
      for k in range(3):
        acc_v[3 * g + k, pl.ds(j * LANES, LANES)] = accs[3 * j + k]

  pltpu.sync_copy(acc_v, out_hbm.at[wid])


def kernel(pos_pair, neg_pair, emb):
  B = pos_pair.shape[0]
  per_w = B // NW
  n_chunks = per_w // CHUNK
  idx = jnp.stack([pos_pair[:, 0], pos_pair[:, 1],
                   neg_pair[:, 0], neg_pair[:, 1]]).astype(jnp.int32)
  idx = idx.reshape(4, NW, n_chunks, CHUNK).transpose(1, 0, 2, 3)

  mesh = plsc.VectorSubcoreMesh(core_axis_name="c", subcore_axis_name="s",
                                num_cores=NC, num_subcores=NS)
  scratch = [pltpu.VMEM((4, n_chunks, CHUNK), jnp.int32)]
  scratch += [pltpu.VMEM((CHUNK, 128), jnp.float32) for _ in range(2 * NBUF)]
  scratch += [pltpu.VMEM((6, 128), jnp.float32)]
  scratch += [pltpu.SemaphoreType.DMA for _ in range(NBUF)]
  partials = pl.kernel(
      _sc_body,
      out_type=jax.ShapeDtypeStruct((NW, 6, 128), jnp.float32),
      mesh=mesh,
      scratch_types=scratch,
  )(idx, emb)

  p = jnp.sum(partials, axis=0)
  eps = jnp.float32(1e-8)

  def cos(num, so, sd):
    return num / (jnp.maximum(jnp.sqrt(so), eps) * jnp.maximum(jnp.sqrt(sd), eps))

  pos_dist = cos(p[0], p[1], p[2])
  neg_dist = cos(p[3], p[4], p[5])
  return (pos_dist, neg_dist)
